# baseline (speedup 1.0000x reference)
import functools

import jax
import jax.numpy as jnp
from jax import lax
from jax.experimental import pallas as pl
from jax.experimental.pallas import tpu as pltpu
from jax.experimental.pallas import tpu_sc as plsc

N = 10000
E = 320000
D_IN = 128
H1 = 64
H2 = 32
NC, NS = 2, 16
NPAD = 10240
RPS = NPAD // NS
CHUNK = 128
CPT = 160
TCH = NS * CPT
EPAD = TCH * CHUNK
DEGW = 8
NB = 2


def _mm_body(x_ref, w_ref, o_ref):
    o_ref[...] = jnp.dot(x_ref[...], w_ref[...],
                         preferred_element_type=jnp.float32)


def _matmul(x, w, blk=1024):
    m, k = x.shape
    n = w.shape[1]
    return pl.pallas_call(
        _mm_body,
        grid=(m // blk,),
        in_specs=[pl.BlockSpec((blk, k), lambda i: (i, 0)),
                  pl.BlockSpec((k, n), lambda i: (0, 0))],
        out_specs=pl.BlockSpec((blk, n), lambda i: (i, 0)),
        out_shape=jax.ShapeDtypeStruct((m, n), jnp.float32),
    )(x, w)


_MESH = plsc.VectorSubcoreMesh(core_axis_name="c", subcore_axis_name="s",
                               num_cores=NC, num_subcores=NS)


@functools.partial(
    pl.kernel,
    out_type=[jax.ShapeDtypeStruct((NPAD, H1), jnp.float32),
              jax.ShapeDtypeStruct((NPAD, DEGW), jnp.float32)],
    mesh=_MESH,
    scratch_types=[
        pltpu.VMEM((CPT, CHUNK), jnp.int32),
        pltpu.VMEM((CPT, CHUNK), jnp.int32),
        pltpu.VMEM((NB, CHUNK, H1), jnp.float32),
        pltpu.VMEM((CHUNK, DEGW), jnp.float32),
        pltpu.VMEM_SHARED((NPAD, H1), jnp.float32),
        pltpu.VMEM_SHARED((NPAD, DEGW), jnp.float32),
        pltpu.SemaphoreType.DMA((NB,)),
        pltpu.SemaphoreType.DMA((NB,)),
        pltpu.SemaphoreType.DMA((NB,)),
        pltpu.SemaphoreType.DMA((3,)),
    ],
    compiler_params=pltpu.CompilerParams(use_tc_tiling_on_sc=False),
)
def _seg_sum1(edges, table, z_w, z_d, ones_hbm,
              acc_out, deg_out,
              src_idx, dst_idx, rows, ones_v, acc_sh, deg_sh,
              gsem, ssem, dsem, isem):
    c = lax.axis_index("c")
    s = lax.axis_index("s")

    @pl.when(c == 0)
    def _():
        r0 = s * RPS
        iz = pltpu.async_copy(z_w, acc_sh.at[pl.ds(r0, RPS)], isem.at[0])
        izd = pltpu.async_copy(z_d, deg_sh.at[pl.ds(r0, RPS)], isem.at[1])
        io = pltpu.async_copy(ones_hbm, ones_v, isem.at[2])
        pltpu.sync_copy(edges.at[0, pl.ds(s * CPT, CPT)], src_idx)
        pltpu.sync_copy(edges.at[1, pl.ds(s * CPT, CPT)], dst_idx)
        for b in range(NB):
            pltpu.async_copy(table.at[src_idx.at[b]], rows.at[b], gsem.at[b])
        iz.wait()
        izd.wait()
        io.wait()
        plsc.subcore_barrier()

        def group(g, carry):
            base = g * NB
            for b in range(NB):
                j = base + b
                pltpu.make_async_copy(table.at[src_idx.at[j]], rows.at[b],
                                      gsem.at[b]).wait()
                pltpu.async_copy(rows.at[b], acc_sh.at[dst_idx.at[j]],
                                 ssem.at[b], add=True)
                pltpu.async_copy(ones_v, deg_sh.at[dst_idx.at[j]],
                                 dsem.at[b], add=True)
            for b in range(NB):
                j = base + b
                pltpu.make_async_copy(rows.at[b], acc_sh.at[dst_idx.at[j]],
                                      ssem.at[b]).wait()
                pltpu.make_async_copy(ones_v, deg_sh.at[dst_idx.at[j]],
                                      dsem.at[b]).wait()

                @pl.when(j + NB < CPT)
                def _():
                    pltpu.async_copy(table.at[src_idx.at[j + NB]],
                                     rows.at[b], gsem.at[b])
            return carry

        lax.fori_loop(0, CPT // NB, group, 0)
        plsc.subcore_barrier()
        pltpu.sync_copy(acc_sh.at[pl.ds(r0, RPS)], acc_out.at[pl.ds(r0, RPS)])
        pltpu.sync_copy(deg_sh.at[pl.ds(r0, RPS)], deg_out.at[pl.ds(r0, RPS)])


@functools.partial(
    pl.kernel,
    out_type=jax.ShapeDtypeStruct((NPAD, H2), jnp.float32),
    mesh=_MESH,
    scratch_types=[
        pltpu.VMEM((CPT, CHUNK), jnp.int32),
        pltpu.VMEM((CPT, CHUNK), jnp.int32),
        pltpu.VMEM((NB, CHUNK, H2), jnp.float32),
        pltpu.VMEM_SHARED((NPAD, H2), jnp.float32),
        pltpu.SemaphoreType.DMA((NB,)),
        pltpu.SemaphoreType.DMA((NB,)),
        pltpu.SemaphoreType.DMA,
    ],
    compiler_params=pltpu.CompilerParams(use_tc_tiling_on_sc=False),
)
def _seg_sum2(edges, table, z_w,
              acc_out,
              src_idx, dst_idx, rows, acc_sh, gsem, ssem, isem):
    c = lax.axis_index("c")
    s = lax.axis_index("s")

    @pl.when(c == 0)
    def _():
        r0 = s * RPS
        iz = pltpu.async_copy(z_w, acc_sh.at[pl.ds(r0, RPS)], isem)
        pltpu.sync_copy(edges.at[0, pl.ds(s * CPT, CPT)], src_idx)
        pltpu.sync_copy(edges.at[1, pl.ds(s * CPT, CPT)], dst_idx)
        for b in range(NB):
            pltpu.async_copy(table.at[src_idx.at[b]], rows.at[b], gsem.at[b])
        iz.wait()
        plsc.subcore_barrier()

        def group(g, carry):
            base = g * NB
            for b in range(NB):
                j = base + b
                pltpu.make_async_copy(table.at[src_idx.at[j]], rows.at[b],
                                      gsem.at[b]).wait()
                pltpu.async_copy(rows.at[b], acc_sh.at[dst_idx.at[j]],
                                 ssem.at[b], add=True)
            for b in range(NB):
                j = base + b
                pltpu.make_async_copy(rows.at[b], acc_sh.at[dst_idx.at[j]],
                                      ssem.at[b]).wait()

                @pl.when(j + NB < CPT)
                def _():
                    pltpu.async_copy(table.at[src_idx.at[j + NB]],
                                     rows.at[b], gsem.at[b])
            return carry

        lax.fori_loop(0, CPT // NB, group, 0)
        plsc.subcore_barrier()
        pltpu.sync_copy(acc_sh.at[pl.ds(r0, RPS)], acc_out.at[pl.ds(r0, RPS)])


def _mid_body(acc_ref, deg_ref, w_ref, b_ref, o_ref):
    i = pl.program_id(0)
    a = acc_ref[...]
    deg = jnp.maximum(deg_ref[...][:, 0], 1.0)
    h = a / deg[:, None] + b_ref[...]
    h = jnp.maximum(h, 0.0)
    blk = h.shape[0]
    row = i * blk + lax.broadcasted_iota(jnp.int32, (blk, 1), 0)
    h = jnp.where(row < N, h, 0.0)
    o_ref[...] = jnp.dot(h, w_ref[...], preferred_element_type=jnp.float32)


def _mid(acc, degw, w, b, blk=1024):
    n_in = acc.shape[1]
    n_out = w.shape[1]
    return pl.pallas_call(
        _mid_body,
        grid=(NPAD // blk,),
        in_specs=[pl.BlockSpec((blk, n_in), lambda i: (i, 0)),
                  pl.BlockSpec((blk, DEGW), lambda i: (i, 0)),
                  pl.BlockSpec((n_in, n_out), lambda i: (0, 0)),
                  pl.BlockSpec((1, n_in), lambda i: (0, 0))],
        out_specs=pl.BlockSpec((blk, n_out), lambda i: (i, 0)),
        out_shape=jax.ShapeDtypeStruct((NPAD, n_out), jnp.float32),
    )(acc, degw, w, b.reshape(1, n_in))


def _head_body(acc_ref, deg_ref, b_ref, wd_ref, bd_ref, o_ref):
    a = acc_ref[...]
    deg = jnp.maximum(deg_ref[...][:, 0], 1.0)
    h = a / deg[:, None] + b_ref[...]
    h = jnp.maximum(h, 0.0)
    pooled = jnp.mean(h, axis=1, keepdims=True)
    z = pooled * wd_ref[...] + bd_ref[...]
    o_ref[...] = 1.0 / (1.0 + jnp.exp(-z))


def _head(acc, degw, b, wd, bd, blk=1024):
    return pl.pallas_call(
        _head_body,
        grid=(NPAD // blk,),
        in_specs=[pl.BlockSpec((blk, H2), lambda i: (i, 0)),
                  pl.BlockSpec((blk, DEGW), lambda i: (i, 0)),
                  pl.BlockSpec((1, H2), lambda i: (0, 0)),
                  pl.BlockSpec((1, 1), lambda i: (0, 0)),
                  pl.BlockSpec((1, 1), lambda i: (0, 0))],
        out_specs=pl.BlockSpec((blk, 1), lambda i: (i, 0)),
        out_shape=jax.ShapeDtypeStruct((NPAD, 1), jnp.float32),
    )(acc, degw, b.reshape(1, H2), wd.reshape(1, 1), bd.reshape(1, 1))


def kernel(x, edge_index, W1, b1, W2, b2, Wd, bd):
    x = x.astype(jnp.float32)
    ei = edge_index.astype(jnp.int32)
    pad = EPAD - E
    src = jnp.concatenate([ei[0], jnp.full((pad,), N, jnp.int32)])
    dst = jnp.concatenate([ei[1], jnp.full((pad,), NPAD - 1, jnp.int32)])
    edges = jnp.stack([src, dst]).reshape(2, TCH, CHUNK)

    x_pad = jnp.pad(x, ((0, NPAD - N), (0, 0)))
    z1 = jnp.zeros((RPS, H1), jnp.float32)
    z2 = jnp.zeros((RPS, H2), jnp.float32)
    zd = jnp.zeros((RPS, DEGW), jnp.float32)
    ones = jnp.ones((CHUNK, DEGW), jnp.float32)

    y1 = _matmul(x_pad, W1)
    acc1, degw = _seg_sum1(edges, y1, z1, zd, ones)
    y2 = _mid(acc1, degw, W2, b1)
    acc2 = _seg_sum2(edges, y2, z2)
    out = _head(acc2, degw, b2, Wd, bd)
    return out[:N]


# speedup vs baseline: 1.3274x; 1.3274x over previous
import functools

import jax
import jax.numpy as jnp
from jax import lax
from jax.experimental import pallas as pl
from jax.experimental.pallas import tpu as pltpu
from jax.experimental.pallas import tpu_sc as plsc

N = 10000
E = 320000
D_IN = 128
H1 = 64
H2 = 32
NC, NS = 2, 16
NPAD = 10240
RPS = NPAD // NS
CHUNK = 128
CPT0 = 120
CPT1 = 40
TCH = NS * (CPT0 + CPT1)
NCH0 = NS * CPT0
EPAD = TCH * CHUNK
DEGW = 8
NB = 4


def _mm_body(x_ref, w_ref, o_ref):
    o_ref[...] = jnp.dot(x_ref[...], w_ref[...],
                         preferred_element_type=jnp.float32)


def _matmul(x, w, blk=1024):
    m, k = x.shape
    n = w.shape[1]
    return pl.pallas_call(
        _mm_body,
        grid=(m // blk,),
        in_specs=[pl.BlockSpec((blk, k), lambda i: (i, 0)),
                  pl.BlockSpec((k, n), lambda i: (0, 0))],
        out_specs=pl.BlockSpec((blk, n), lambda i: (i, 0)),
        out_shape=jax.ShapeDtypeStruct((m, n), jnp.float32),
    )(x, w)


_MESH = plsc.VectorSubcoreMesh(core_axis_name="c", subcore_axis_name="s",
                               num_cores=NC, num_subcores=NS)


@functools.partial(
    pl.kernel,
    out_type=[jax.ShapeDtypeStruct((NC, NPAD, H1), jnp.float32),
              jax.ShapeDtypeStruct((NC, NPAD, DEGW), jnp.float32)],
    mesh=_MESH,
    scratch_types=[
        pltpu.VMEM((CPT0, CHUNK), jnp.int32),
        pltpu.VMEM((CPT0, CHUNK), jnp.int32),
        pltpu.VMEM((NB, CHUNK, H1), jnp.float32),
        pltpu.VMEM((CHUNK, DEGW), jnp.float32),
        pltpu.VMEM_SHARED((NPAD, H1), jnp.float32),
        pltpu.VMEM_SHARED((NPAD, DEGW), jnp.float32),
        pltpu.SemaphoreType.DMA((NB,)),
        pltpu.SemaphoreType.DMA((NB,)),
        pltpu.SemaphoreType.DMA((NB,)),
        pltpu.SemaphoreType.DMA((5,)),
    ],
    compiler_params=pltpu.CompilerParams(use_tc_tiling_on_sc=False),
)
def _seg_sum1(edges, table, z_w, z_d, ones_hbm,
              acc_out, deg_out,
              src_idx, dst_idx, rows, ones_v, acc_sh, deg_sh,
              gsem, ssem, dsem, isem):
    c = lax.axis_index("c")
    s = lax.axis_index("s")
    cpt = jnp.where(c == 0, CPT0, CPT1)
    r0 = s * RPS
    iz = pltpu.async_copy(z_w, acc_sh.at[pl.ds(r0, RPS)], isem.at[0])
    izd = pltpu.async_copy(z_d, deg_sh.at[pl.ds(r0, RPS)], isem.at[1])
    io = pltpu.async_copy(ones_hbm, ones_v, isem.at[2])

    @pl.when(c == 0)
    def _():
        pltpu.async_copy(edges.at[0, pl.ds(s * CPT0, CPT0)],
                         src_idx.at[pl.ds(0, CPT0)], isem.at[3])
        pltpu.async_copy(edges.at[1, pl.ds(s * CPT0, CPT0)],
                         dst_idx.at[pl.ds(0, CPT0)], isem.at[4])
        pltpu.make_async_copy(edges.at[0, pl.ds(0, CPT0)],
                              src_idx.at[pl.ds(0, CPT0)], isem.at[3]).wait()
        pltpu.make_async_copy(edges.at[1, pl.ds(0, CPT0)],
                              dst_idx.at[pl.ds(0, CPT0)], isem.at[4]).wait()

    @pl.when(c == 1)
    def _():
        pltpu.async_copy(edges.at[0, pl.ds(NCH0 + s * CPT1, CPT1)],
                         src_idx.at[pl.ds(0, CPT1)], isem.at[3])
        pltpu.async_copy(edges.at[1, pl.ds(NCH0 + s * CPT1, CPT1)],
                         dst_idx.at[pl.ds(0, CPT1)], isem.at[4])
        pltpu.make_async_copy(edges.at[0, pl.ds(0, CPT1)],
                              src_idx.at[pl.ds(0, CPT1)], isem.at[3]).wait()
        pltpu.make_async_copy(edges.at[1, pl.ds(0, CPT1)],
                              dst_idx.at[pl.ds(0, CPT1)], isem.at[4]).wait()

    for b in range(NB):
        pltpu.async_copy(table.at[src_idx.at[b]], rows.at[b], gsem.at[b])
    iz.wait()
    izd.wait()
    io.wait()
    plsc.subcore_barrier()

    def group(g, carry):
        base = g * NB
        for b in range(NB):
            j = base + b
            pltpu.make_async_copy(table.at[src_idx.at[j]], rows.at[b],
                                  gsem.at[b]).wait()
            pltpu.async_copy(rows.at[b], acc_sh.at[dst_idx.at[j]],
                             ssem.at[b], add=True)
            pltpu.async_copy(ones_v, deg_sh.at[dst_idx.at[j]],
                             dsem.at[b], add=True)
        for b in range(NB):
            j = base + b
            pltpu.make_async_copy(rows.at[b], acc_sh.at[dst_idx.at[j]],
                                  ssem.at[b]).wait()
            pltpu.make_async_copy(ones_v, deg_sh.at[dst_idx.at[j]],
                                  dsem.at[b]).wait()

            @pl.when(j + NB < cpt)
            def _():
                pltpu.async_copy(table.at[src_idx.at[j + NB]],
                                 rows.at[b], gsem.at[b])
        return carry

    lax.fori_loop(0, cpt // NB, group, 0)
    plsc.subcore_barrier()
    pltpu.async_copy(acc_sh.at[pl.ds(r0, RPS)],
                     acc_out.at[c, pl.ds(r0, RPS)], isem.at[0])
    pltpu.async_copy(deg_sh.at[pl.ds(r0, RPS)],
                     deg_out.at[c, pl.ds(r0, RPS)], isem.at[1])
    pltpu.make_async_copy(acc_sh.at[pl.ds(r0, RPS)],
                          acc_out.at[c, pl.ds(r0, RPS)], isem.at[0]).wait()
    pltpu.make_async_copy(deg_sh.at[pl.ds(r0, RPS)],
                          deg_out.at[c, pl.ds(r0, RPS)], isem.at[1]).wait()


@functools.partial(
    pl.kernel,
    out_type=jax.ShapeDtypeStruct((NC, NPAD, H2), jnp.float32),
    mesh=_MESH,
    scratch_types=[
        pltpu.VMEM((CPT0, CHUNK), jnp.int32),
        pltpu.VMEM((CPT0, CHUNK), jnp.int32),
        pltpu.VMEM((NB, CHUNK, H2), jnp.float32),
        pltpu.VMEM_SHARED((NPAD, H2), jnp.float32),
        pltpu.SemaphoreType.DMA((NB,)),
        pltpu.SemaphoreType.DMA((NB,)),
        pltpu.SemaphoreType.DMA((3,)),
    ],
    compiler_params=pltpu.CompilerParams(use_tc_tiling_on_sc=False),
)
def _seg_sum2(edges, table, z_w,
              acc_out,
              src_idx, dst_idx, rows, acc_sh, gsem, ssem, isem):
    c = lax.axis_index("c")
    s = lax.axis_index("s")
    cpt = jnp.where(c == 0, CPT0, CPT1)
    r0 = s * RPS
    iz = pltpu.async_copy(z_w, acc_sh.at[pl.ds(r0, RPS)], isem.at[0])

    @pl.when(c == 0)
    def _():
        pltpu.async_copy(edges.at[0, pl.ds(s * CPT0, CPT0)],
                         src_idx.at[pl.ds(0, CPT0)], isem.at[1])
        pltpu.async_copy(edges.at[1, pl.ds(s * CPT0, CPT0)],
                         dst_idx.at[pl.ds(0, CPT0)], isem.at[2])
        pltpu.make_async_copy(edges.at[0, pl.ds(0, CPT0)],
                              src_idx.at[pl.ds(0, CPT0)], isem.at[1]).wait()
        pltpu.make_async_copy(edges.at[1, pl.ds(0, CPT0)],
                              dst_idx.at[pl.ds(0, CPT0)], isem.at[2]).wait()

    @pl.when(c == 1)
    def _():
        pltpu.async_copy(edges.at[0, pl.ds(NCH0 + s * CPT1, CPT1)],
                         src_idx.at[pl.ds(0, CPT1)], isem.at[1])
        pltpu.async_copy(edges.at[1, pl.ds(NCH0 + s * CPT1, CPT1)],
                         dst_idx.at[pl.ds(0, CPT1)], isem.at[2])
        pltpu.make_async_copy(edges.at[0, pl.ds(0, CPT1)],
                              src_idx.at[pl.ds(0, CPT1)], isem.at[1]).wait()
        pltpu.make_async_copy(edges.at[1, pl.ds(0, CPT1)],
                              dst_idx.at[pl.ds(0, CPT1)], isem.at[2]).wait()

    for b in range(NB):
        pltpu.async_copy(table.at[src_idx.at[b]], rows.at[b], gsem.at[b])
    iz.wait()
    plsc.subcore_barrier()

    def group(g, carry):
        base = g * NB
        for b in range(NB):
            j = base + b
            pltpu.make_async_copy(table.at[src_idx.at[j]], rows.at[b],
                                  gsem.at[b]).wait()
            pltpu.async_copy(rows.at[b], acc_sh.at[dst_idx.at[j]],
                             ssem.at[b], add=True)
        for b in range(NB):
            j = base + b
            pltpu.make_async_copy(rows.at[b], acc_sh.at[dst_idx.at[j]],
                                  ssem.at[b]).wait()

            @pl.when(j + NB < cpt)
            def _():
                pltpu.async_copy(table.at[src_idx.at[j + NB]],
                                 rows.at[b], gsem.at[b])
        return carry

    lax.fori_loop(0, cpt // NB, group, 0)
    plsc.subcore_barrier()
    pltpu.sync_copy(acc_sh.at[pl.ds(r0, RPS)], acc_out.at[c, pl.ds(r0, RPS)])


def _mid_body(acc_ref, deg_ref, w_ref, b_ref, o_ref):
    i = pl.program_id(0)
    a = acc_ref[...]
    d = deg_ref[...]
    deg = jnp.maximum(d[0, :, 0] + d[1, :, 0], 1.0)
    h = (a[0] + a[1]) / deg[:, None] + b_ref[...]
    h = jnp.maximum(h, 0.0)
    blk = h.shape[0]
    row = i * blk + lax.broadcasted_iota(jnp.int32, (blk, 1), 0)
    h = jnp.where(row < N, h, 0.0)
    o_ref[...] = jnp.dot(h, w_ref[...], preferred_element_type=jnp.float32)


def _mid(acc, degw, w, b, blk=1024):
    n_in = acc.shape[2]
    n_out = w.shape[1]
    return pl.pallas_call(
        _mid_body,
        grid=(NPAD // blk,),
        in_specs=[pl.BlockSpec((NC, blk, n_in), lambda i: (0, i, 0)),
                  pl.BlockSpec((NC, blk, DEGW), lambda i: (0, i, 0)),
                  pl.BlockSpec((n_in, n_out), lambda i: (0, 0)),
                  pl.BlockSpec((1, n_in), lambda i: (0, 0))],
        out_specs=pl.BlockSpec((blk, n_out), lambda i: (i, 0)),
        out_shape=jax.ShapeDtypeStruct((NPAD, n_out), jnp.float32),
    )(acc, degw, w, b.reshape(1, n_in))


def _head_body(acc_ref, deg_ref, b_ref, wd_ref, bd_ref, o_ref):
    a = acc_ref[...]
    d = deg_ref[...]
    deg = jnp.maximum(d[0, :, 0] + d[1, :, 0], 1.0)
    h = (a[0] + a[1]) / deg[:, None] + b_ref[...]
    h = jnp.maximum(h, 0.0)
    pooled = jnp.mean(h, axis=1, keepdims=True)
    z = pooled * wd_ref[...] + bd_ref[...]
    o_ref[...] = 1.0 / (1.0 + jnp.exp(-z))


def _head(acc, degw, b, wd, bd, blk=1024):
    return pl.pallas_call(
        _head_body,
        grid=(NPAD // blk,),
        in_specs=[pl.BlockSpec((NC, blk, H2), lambda i: (0, i, 0)),
                  pl.BlockSpec((NC, blk, DEGW), lambda i: (0, i, 0)),
                  pl.BlockSpec((1, H2), lambda i: (0, 0)),
                  pl.BlockSpec((1, 1), lambda i: (0, 0)),
                  pl.BlockSpec((1, 1), lambda i: (0, 0))],
        out_specs=pl.BlockSpec((blk, 1), lambda i: (i, 0)),
        out_shape=jax.ShapeDtypeStruct((NPAD, 1), jnp.float32),
    )(acc, degw, b.reshape(1, H2), wd.reshape(1, 1), bd.reshape(1, 1))


def kernel(x, edge_index, W1, b1, W2, b2, Wd, bd):
    x = x.astype(jnp.float32)
    ei = edge_index.astype(jnp.int32)
    pad = EPAD - E
    src = jnp.concatenate([ei[0], jnp.full((pad,), N, jnp.int32)])
    dst = jnp.concatenate([ei[1], jnp.full((pad,), NPAD - 1, jnp.int32)])
    edges = jnp.stack([src, dst]).reshape(2, TCH, CHUNK)

    x_pad = jnp.pad(x, ((0, NPAD - N), (0, 0)))
    z1 = jnp.zeros((RPS, H1), jnp.float32)
    z2 = jnp.zeros((RPS, H2), jnp.float32)
    zd = jnp.zeros((RPS, DEGW), jnp.float32)
    ones = jnp.ones((CHUNK, DEGW), jnp.float32)

    y1 = _matmul(x_pad, W1)
    acc1, degw = _seg_sum1(edges, y1, z1, zd, ones)
    y2 = _mid(acc1, degw, W2, b1)
    acc2 = _seg_sum2(edges, y2, z2)
    out = _head(acc2, degw, b2, Wd, bd)
    return out[:N]
